# baseline (device time: 82187 ns/iter reference)
import jax
import jax.numpy as jnp
from jax import lax
from jax.experimental import pallas as pl
from jax.experimental.pallas import tpu as pltpu

N_DEV = 4
P = 4


def kernel(x, w_mat):
    m, k_per = x.shape
    _, n = w_mat.shape
    m_out = m // N_DEV
    nh = n // 2
    rp = m_out // P

    def body(x_hbm, w_hbm, out_ref,
             x_v, w_v, in_sems,
             sA_r, sA_l, sBi_r, sBi_l, sBd_r, sBd_l,
             rA_l, rA_r, rBi_l, rBi_r, rBd_l, rBd_r,
             s_sems, r_sems, a_s_r, a_s_l, a_r_l, a_r_r,
             dep_s_r, dep_s_l, dep_r_l, dep_r_r):
        my = lax.axis_index("i")
        left = lax.rem(my + N_DEV - 1, N_DEV)
        right = lax.rem(my + 1, N_DEV)

        cp_x = pltpu.make_async_copy(x_hbm, x_v, in_sems.at[0])
        cp_w = pltpu.make_async_copy(w_hbm, w_v, in_sems.at[1])
        cp_x.start()
        cp_w.start()

        barrier_sem = pltpu.get_barrier_semaphore()
        for nbr in (left, right):
            pl.semaphore_signal(
                barrier_sem, inc=1,
                device_id=(nbr,), device_id_type=pl.DeviceIdType.MESH,
            )
        pl.semaphore_wait(barrier_sem, 2)
        cp_x.wait()
        cp_w.wait()

        def partial(c, j=None):
            if j is None:
                xs = x_v[pl.ds(c * m_out, m_out), :]
            else:
                xs = x_v[pl.ds(c * m_out + j * rp, rp), :]
            return jnp.dot(xs, w_v[:, :], preferred_element_type=jnp.float32)

        def copy(src, dst, ssem, rsem, dev):
            return pltpu.make_async_remote_copy(
                src_ref=src, dst_ref=dst, send_sem=ssem, recv_sem=rsem,
                device_id=(dev,), device_id_type=pl.DeviceIdType.MESH,
            )

        snd_A_r = [
            copy(sA_r.at[pl.ds(j * rp, rp), :], rA_l.at[pl.ds(j * rp, rp), :],
                 a_s_r.at[j], a_r_l.at[j], right)
            for j in range(P)
        ]
        snd_A_l = [
            copy(sA_l.at[pl.ds(j * rp, rp), :], rA_r.at[pl.ds(j * rp, rp), :],
                 a_s_l.at[j], a_r_r.at[j], left)
            for j in range(P)
        ]
        snd_Bi_r = copy(sBi_r, rBi_l, s_sems.at[0], r_sems.at[0], right)
        snd_Bi_l = copy(sBi_l, rBi_r, s_sems.at[1], r_sems.at[1], left)
        snd_Bd_r = [
            copy(sBd_r.at[pl.ds(j * rp, rp), :], rBd_l.at[pl.ds(j * rp, rp), :],
                 dep_s_r.at[j], dep_r_l.at[j], right)
            for j in range(P)
        ]
        snd_Bd_l = [
            copy(sBd_l.at[pl.ds(j * rp, rp), :], rBd_r.at[pl.ds(j * rp, rp), :],
                 dep_s_l.at[j], dep_r_r.at[j], left)
            for j in range(P)
        ]
        rcv_A_l = [
            copy(rA_l.at[pl.ds(j * rp, rp), :], rA_l.at[pl.ds(j * rp, rp), :],
                 a_s_r.at[j], a_r_l.at[j], left)
            for j in range(P)
        ]
        rcv_A_r = [
            copy(rA_r.at[pl.ds(j * rp, rp), :], rA_r.at[pl.ds(j * rp, rp), :],
                 a_s_l.at[j], a_r_r.at[j], right)
            for j in range(P)
        ]
        rcv_Bi_l = copy(rBi_l, rBi_l, s_sems.at[0], r_sems.at[0], left)
        rcv_Bi_r = copy(rBi_r, rBi_r, s_sems.at[1], r_sems.at[1], right)
        rcv_Bd_l = [
            copy(rBd_l.at[pl.ds(j * rp, rp), :], rBd_l.at[pl.ds(j * rp, rp), :],
                 dep_s_r.at[j], dep_r_l.at[j], left)
            for j in range(P)
        ]
        rcv_Bd_r = [
            copy(rBd_r.at[pl.ds(j * rp, rp), :], rBd_r.at[pl.ds(j * rp, rp), :],
                 dep_s_l.at[j], dep_r_r.at[j], right)
            for j in range(P)
        ]

        diag = lax.rem(my + 2, N_DEV)
        for j in range(P):
            r0, r1 = j * rp, (j + 1) * rp
            g2j = partial(diag, j)
            sA_r[r0:r1, :] = g2j[:, 0:nh]
            sA_l[r0:r1, :] = g2j[:, nh:n]
            snd_A_r[j].start()
            snd_A_l[j].start()

        g1 = partial(right)
        sBi_r[:, :] = g1[:, nh:n]
        snd_Bi_r.start()
        gm1 = partial(left)
        sBi_l[:, :] = gm1[:, 0:nh]
        snd_Bi_l.start()

        for j in range(P):
            r0, r1 = j * rp, (j + 1) * rp
            rcv_A_l[j].wait_recv()
            sBd_r[r0:r1, :] = g1[r0:r1, 0:nh] + rA_l[r0:r1, :]
            snd_Bd_r[j].start()
            rcv_A_r[j].wait_recv()
            sBd_l[r0:r1, :] = gm1[r0:r1, nh:n] + rA_r[r0:r1, :]
            snd_Bd_l[j].start()

        g0 = partial(my)

        rcv_Bi_r.wait_recv()
        rcv_Bi_l.wait_recv()
        for j in range(P):
            r0, r1 = j * rp, (j + 1) * rp
            rcv_Bd_l[j].wait_recv()
            out_ref[r0:r1, 0:nh] = jnp.maximum(
                g0[r0:r1, 0:nh] + rBi_r[r0:r1, :] + rBd_l[r0:r1, :], 0.0)
            rcv_Bd_r[j].wait_recv()
            out_ref[r0:r1, nh:n] = jnp.maximum(
                g0[r0:r1, nh:n] + rBi_l[r0:r1, :] + rBd_r[r0:r1, :], 0.0)

        for d in (*snd_A_r, *snd_A_l, snd_Bi_r, snd_Bi_l,
                  *snd_Bd_r, *snd_Bd_l):
            d.wait_send()

    half = lambda: pltpu.VMEM((m_out, nh), jnp.float32)
    return pl.pallas_call(
        body,
        out_shape=jax.ShapeDtypeStruct((m_out, n), jnp.float32),
        in_specs=[
            pl.BlockSpec(memory_space=pl.ANY),
            pl.BlockSpec(memory_space=pl.ANY),
        ],
        out_specs=pl.BlockSpec(memory_space=pltpu.VMEM),
        scratch_shapes=[
            pltpu.VMEM((m, k_per), jnp.float32),
            pltpu.VMEM((k_per, n), jnp.float32),
            pltpu.SemaphoreType.DMA((2,)),
            half(), half(), half(), half(), half(), half(),
            half(), half(), half(), half(), half(), half(),
            pltpu.SemaphoreType.DMA((2,)),
            pltpu.SemaphoreType.DMA((2,)),
            pltpu.SemaphoreType.DMA((P,)),
            pltpu.SemaphoreType.DMA((P,)),
            pltpu.SemaphoreType.DMA((P,)),
            pltpu.SemaphoreType.DMA((P,)),
            pltpu.SemaphoreType.DMA((P,)),
            pltpu.SemaphoreType.DMA((P,)),
            pltpu.SemaphoreType.DMA((P,)),
            pltpu.SemaphoreType.DMA((P,)),
        ],
        compiler_params=pltpu.CompilerParams(
            collective_id=0,
            vmem_limit_bytes=64 * 1024 * 1024,
        ),
    )(x, w_mat)


# device time: 79595 ns/iter; 1.0326x vs baseline; 1.0326x over previous
import jax
import jax.numpy as jnp
from jax import lax
from jax.experimental import pallas as pl
from jax.experimental.pallas import tpu as pltpu

N_DEV = 4
P = 4


def kernel(x, w_mat):
    m, k_per = x.shape
    _, n = w_mat.shape
    m_out = m // N_DEV
    nh = n // 2
    rp = m_out // P

    def body(x_ref, w_ref, out_ref,
             sA_r, sA_l, sBi_r, sBi_l, sBd_r, sBd_l,
             rA_l, rA_r, rBi_l, rBi_r, rBd_l, rBd_r,
             s_sems, r_sems, a_s_r, a_s_l, a_r_l, a_r_r,
             dep_s_r, dep_s_l, dep_r_l, dep_r_r):
        my = lax.axis_index("i")
        left = lax.rem(my + N_DEV - 1, N_DEV)
        right = lax.rem(my + 1, N_DEV)

        barrier_sem = pltpu.get_barrier_semaphore()
        for nbr in (left, right):
            pl.semaphore_signal(
                barrier_sem, inc=1,
                device_id=(nbr,), device_id_type=pl.DeviceIdType.MESH,
            )
        pl.semaphore_wait(barrier_sem, 2)

        def partial(c, j=None):
            if j is None:
                xs = x_ref[pl.ds(c * m_out, m_out), :]
            else:
                xs = x_ref[pl.ds(c * m_out + j * rp, rp), :]
            return jnp.dot(xs, w_ref[:, :], preferred_element_type=jnp.float32)

        def copy(src, dst, ssem, rsem, dev):
            return pltpu.make_async_remote_copy(
                src_ref=src, dst_ref=dst, send_sem=ssem, recv_sem=rsem,
                device_id=(dev,), device_id_type=pl.DeviceIdType.MESH,
            )

        snd_A_r = [
            copy(sA_r.at[pl.ds(j * rp, rp), :], rA_l.at[pl.ds(j * rp, rp), :],
                 a_s_r.at[j], a_r_l.at[j], right)
            for j in range(P)
        ]
        snd_A_l = [
            copy(sA_l.at[pl.ds(j * rp, rp), :], rA_r.at[pl.ds(j * rp, rp), :],
                 a_s_l.at[j], a_r_r.at[j], left)
            for j in range(P)
        ]
        snd_Bi_r = copy(sBi_r, rBi_l, s_sems.at[0], r_sems.at[0], right)
        snd_Bi_l = copy(sBi_l, rBi_r, s_sems.at[1], r_sems.at[1], left)
        snd_Bd_r = [
            copy(sBd_r.at[pl.ds(j * rp, rp), :], rBd_l.at[pl.ds(j * rp, rp), :],
                 dep_s_r.at[j], dep_r_l.at[j], right)
            for j in range(P)
        ]
        snd_Bd_l = [
            copy(sBd_l.at[pl.ds(j * rp, rp), :], rBd_r.at[pl.ds(j * rp, rp), :],
                 dep_s_l.at[j], dep_r_r.at[j], left)
            for j in range(P)
        ]
        rcv_A_l = [
            copy(rA_l.at[pl.ds(j * rp, rp), :], rA_l.at[pl.ds(j * rp, rp), :],
                 a_s_r.at[j], a_r_l.at[j], left)
            for j in range(P)
        ]
        rcv_A_r = [
            copy(rA_r.at[pl.ds(j * rp, rp), :], rA_r.at[pl.ds(j * rp, rp), :],
                 a_s_l.at[j], a_r_r.at[j], right)
            for j in range(P)
        ]
        rcv_Bi_l = copy(rBi_l, rBi_l, s_sems.at[0], r_sems.at[0], left)
        rcv_Bi_r = copy(rBi_r, rBi_r, s_sems.at[1], r_sems.at[1], right)
        rcv_Bd_l = [
            copy(rBd_l.at[pl.ds(j * rp, rp), :], rBd_l.at[pl.ds(j * rp, rp), :],
                 dep_s_r.at[j], dep_r_l.at[j], left)
            for j in range(P)
        ]
        rcv_Bd_r = [
            copy(rBd_r.at[pl.ds(j * rp, rp), :], rBd_r.at[pl.ds(j * rp, rp), :],
                 dep_s_l.at[j], dep_r_r.at[j], right)
            for j in range(P)
        ]

        diag = lax.rem(my + 2, N_DEV)
        for j in range(P):
            r0, r1 = j * rp, (j + 1) * rp
            g2j = partial(diag, j)
            sA_r[r0:r1, :] = g2j[:, 0:nh]
            sA_l[r0:r1, :] = g2j[:, nh:n]
            snd_A_r[j].start()
            snd_A_l[j].start()

        g1 = partial(right)
        sBi_r[:, :] = g1[:, nh:n]
        snd_Bi_r.start()
        gm1 = partial(left)
        sBi_l[:, :] = gm1[:, 0:nh]
        snd_Bi_l.start()

        for j in range(P):
            r0, r1 = j * rp, (j + 1) * rp
            rcv_A_l[j].wait_recv()
            sBd_r[r0:r1, :] = g1[r0:r1, 0:nh] + rA_l[r0:r1, :]
            snd_Bd_r[j].start()
            rcv_A_r[j].wait_recv()
            sBd_l[r0:r1, :] = gm1[r0:r1, nh:n] + rA_r[r0:r1, :]
            snd_Bd_l[j].start()

        g0 = partial(my)

        rcv_Bi_r.wait_recv()
        rcv_Bi_l.wait_recv()
        for j in range(P):
            r0, r1 = j * rp, (j + 1) * rp
            rcv_Bd_l[j].wait_recv()
            out_ref[r0:r1, 0:nh] = jnp.maximum(
                g0[r0:r1, 0:nh] + rBi_r[r0:r1, :] + rBd_l[r0:r1, :], 0.0)
            rcv_Bd_r[j].wait_recv()
            out_ref[r0:r1, nh:n] = jnp.maximum(
                g0[r0:r1, nh:n] + rBi_l[r0:r1, :] + rBd_r[r0:r1, :], 0.0)

        for d in (*snd_A_r, *snd_A_l, snd_Bi_r, snd_Bi_l,
                  *snd_Bd_r, *snd_Bd_l):
            d.wait_send()

    half = lambda: pltpu.VMEM((m_out, nh), jnp.float32)
    return pl.pallas_call(
        body,
        out_shape=jax.ShapeDtypeStruct((m_out, n), jnp.float32),
        in_specs=[
            pl.BlockSpec(memory_space=pltpu.VMEM),
            pl.BlockSpec(memory_space=pltpu.VMEM),
        ],
        out_specs=pl.BlockSpec(memory_space=pltpu.VMEM),
        scratch_shapes=[
            half(), half(), half(), half(), half(), half(),
            half(), half(), half(), half(), half(), half(),
            pltpu.SemaphoreType.DMA((2,)),
            pltpu.SemaphoreType.DMA((2,)),
            pltpu.SemaphoreType.DMA((P,)),
            pltpu.SemaphoreType.DMA((P,)),
            pltpu.SemaphoreType.DMA((P,)),
            pltpu.SemaphoreType.DMA((P,)),
            pltpu.SemaphoreType.DMA((P,)),
            pltpu.SemaphoreType.DMA((P,)),
            pltpu.SemaphoreType.DMA((P,)),
            pltpu.SemaphoreType.DMA((P,)),
        ],
        compiler_params=pltpu.CompilerParams(collective_id=0),
    )(x, w_mat)
